# Initial kernel scaffold; baseline (speedup 1.0000x reference)
import jax
import jax.numpy as jnp
from jax.experimental import pallas as pl


def kernel(x, edge_index, W1, b1, W2, b2):
    raise NotImplementedError("write your pallas kernel here")


# speedup vs baseline: 16.9478x; 16.9478x over previous
import functools

import jax
import jax.numpy as jnp
from jax import lax
from jax.experimental import pallas as pl
from jax.experimental.pallas import tpu as pltpu
from jax.experimental.pallas import tpu_sc as plsc

_N = 2048
_E = 32768
_HID = 16
_NT = 16
_RPT = _N // _NT
_EPT = _E // _NT
_CHUNK = 128
_NCH = _EPT // _CHUNK


def _mm1_body(x_ref, w_ref, o_ref):
    o_ref[...] = jnp.dot(x_ref[...], w_ref[...],
                         preferred_element_type=jnp.float32)


def _mm2_body(g_ref, w_ref, b_ref, o_ref):
    o_ref[...] = jnp.dot(g_ref[...], w_ref[...],
                         preferred_element_type=jnp.float32) + b_ref[...]


_mm1 = pl.pallas_call(
    _mm1_body,
    out_shape=jax.ShapeDtypeStruct((_N, _HID), jnp.float32),
)

_mm2 = pl.pallas_call(
    _mm2_body,
    out_shape=jax.ShapeDtypeStruct((_N, _N), jnp.float32),
)


def _sc_body(src_hbm, dst_hbm, hpre_hbm, b1_hbm, out_hbm,
             idxs, idxd, rowbuf, hbuf, disbuf, onesbuf, accbuf, b1buf,
             s_deg, s_h, s_acc, s_acc2):
    cid = lax.axis_index("c")
    wid = lax.axis_index("s")

    @pl.when(cid == 0)
    def _core0_work():
        rows = pl.ds(wid * _RPT, _RPT)
        ebase = wid * _EPT

        pltpu.sync_copy(hpre_hbm.at[rows], hbuf)
        pltpu.sync_copy(b1_hbm, b1buf)

        ones = jnp.ones((_HID,), jnp.float32)

        def fill_ones(i, c):
            onesbuf[i, :] = ones
            return c

        lax.fori_loop(0, _RPT, fill_ones, 0)
        pltpu.sync_copy(onesbuf, s_deg.at[rows])
        plsc.subcore_barrier()

        def deg_chunk(j, c):
            pltpu.sync_copy(dst_hbm.at[pl.ds(ebase + j * _CHUNK, _CHUNK)],
                            idxd)
            pltpu.sync_copy(onesbuf, s_deg.at[idxd], add=True)
            return c

        lax.fori_loop(0, _NCH, deg_chunk, 0)
        plsc.subcore_barrier()

        pltpu.sync_copy(s_deg.at[rows], disbuf)

        def mk_dis(i, c):
            d = disbuf[i, :]
            bits = lax.bitcast_convert_type(d, jnp.int32)
            bits = jnp.int32(0x5F3759DF) - lax.shift_right_logical(bits, 1)
            y = lax.bitcast_convert_type(bits, jnp.float32)
            half = 0.5 * d
            y = y * (1.5 - half * y * y)
            y = y * (1.5 - half * y * y)
            y = y * (1.5 - half * y * y)
            disbuf[i, :] = y
            hbuf[i, :] = hbuf[i, :] * y
            return c

        lax.fori_loop(0, _RPT, mk_dis, 0)
        pltpu.sync_copy(hbuf, s_h.at[rows])
        pltpu.sync_copy(hbuf, s_acc.at[rows])
        plsc.subcore_barrier()

        def prop_chunk(j, c, s_to):
            pltpu.sync_copy(src_hbm.at[pl.ds(ebase + j * _CHUNK, _CHUNK)],
                            idxs)
            pltpu.sync_copy(dst_hbm.at[pl.ds(ebase + j * _CHUNK, _CHUNK)],
                            idxd)
            pltpu.sync_copy(s_h.at[idxs], rowbuf)
            pltpu.sync_copy(rowbuf, s_to.at[idxd], add=True)
            return c

        lax.fori_loop(0, _NCH, functools.partial(prop_chunk, s_to=s_acc), 0)
        plsc.subcore_barrier()

        pltpu.sync_copy(s_acc.at[rows], accbuf)
        b1v = b1buf[:]

        def mk_h1(i, c):
            a = accbuf[i, :] * disbuf[i, :] + b1v
            a = jnp.maximum(a, 0.0)
            accbuf[i, :] = a * disbuf[i, :]
            return c

        lax.fori_loop(0, _RPT, mk_h1, 0)
        pltpu.sync_copy(accbuf, s_h.at[rows])
        pltpu.sync_copy(accbuf, s_acc2.at[rows])
        plsc.subcore_barrier()

        lax.fori_loop(0, _NCH, functools.partial(prop_chunk, s_to=s_acc2), 0)
        plsc.subcore_barrier()

        pltpu.sync_copy(s_acc2.at[rows], accbuf)

        def mk_out(i, c):
            accbuf[i, :] = accbuf[i, :] * disbuf[i, :]
            return c

        lax.fori_loop(0, _RPT, mk_out, 0)
        pltpu.sync_copy(accbuf, out_hbm.at[rows])


_sc_prop = functools.partial(
    pl.kernel,
    mesh=plsc.VectorSubcoreMesh(core_axis_name="c", subcore_axis_name="s"),
    compiler_params=pltpu.CompilerParams(use_tc_tiling_on_sc=False),
    out_type=jax.ShapeDtypeStruct((_N, _HID), jnp.float32),
    scratch_types=[
        pltpu.VMEM((_CHUNK,), jnp.int32),
        pltpu.VMEM((_CHUNK,), jnp.int32),
        pltpu.VMEM((_CHUNK, _HID), jnp.float32),
        pltpu.VMEM((_RPT, _HID), jnp.float32),
        pltpu.VMEM((_RPT, _HID), jnp.float32),
        pltpu.VMEM((_RPT, _HID), jnp.float32),
        pltpu.VMEM((_RPT, _HID), jnp.float32),
        pltpu.VMEM((_HID,), jnp.float32),
        pltpu.VMEM_SHARED((_N, _HID), jnp.float32),
        pltpu.VMEM_SHARED((_N, _HID), jnp.float32),
        pltpu.VMEM_SHARED((_N, _HID), jnp.float32),
        pltpu.VMEM_SHARED((_N, _HID), jnp.float32),
    ],
)(_sc_body)


@jax.jit
def kernel(x, edge_index, W1, b1, W2, b2):
    src = edge_index[0]
    dst = edge_index[1]
    hpre = _mm1(x, W1)
    g = _sc_prop(src, dst, hpre, b1)
    out = _mm2(g, W2, b2.reshape(1, _N))
    return out


# speedup vs baseline: 29.5610x; 1.7442x over previous
import functools

import jax
import jax.numpy as jnp
from jax import lax
from jax.experimental import pallas as pl
from jax.experimental.pallas import tpu as pltpu
from jax.experimental.pallas import tpu_sc as plsc

_N = 2048
_E = 32768
_HID = 16
_NT = 16
_RPT = _N // _NT
_EPT = _E // _NT
_CHUNK = 128
_NCH = _EPT // _CHUNK


def _mm1_body(x_ref, w_ref, o_ref):
    o_ref[...] = jnp.dot(x_ref[...], w_ref[...],
                         preferred_element_type=jnp.float32)


def _mm2_body(g_ref, w_ref, b_ref, o_ref):
    o_ref[...] = jnp.dot(g_ref[...], w_ref[...],
                         preferred_element_type=jnp.float32) + b_ref[...]


_mm1 = pl.pallas_call(
    _mm1_body,
    out_shape=jax.ShapeDtypeStruct((_N, _HID), jnp.float32),
)

_mm2 = pl.pallas_call(
    _mm2_body,
    out_shape=jax.ShapeDtypeStruct((_N, _N), jnp.float32),
)


def _sc_body(src_hbm, dst_hbm, hpre_hbm, b1_hbm, out_hbm, *refs):
    sidx = refs[0:_NCH]
    didx = refs[_NCH:2 * _NCH]
    (rowbig, hbuf, disbuf, onesbuf, accbuf, b1buf,
     s_deg, s_h, s_acc, s_acc2, sem_i, sem_g, sem_s) = refs[2 * _NCH:]
    cid = lax.axis_index("c")
    wid = lax.axis_index("s")

    @pl.when(cid == 0)
    def _core0_work():
        rows = pl.ds(wid * _RPT, _RPT)
        ebase = wid * _EPT

        hh = [pltpu.async_copy(
            src_hbm.at[pl.ds(ebase + j * _CHUNK, _CHUNK)], sidx[j], sem_i)
            for j in range(_NCH)]
        hh.append(pltpu.async_copy(hpre_hbm.at[rows], hbuf, sem_i))
        hh.append(pltpu.async_copy(b1_hbm, b1buf, sem_i))

        ones = jnp.ones((_HID,), jnp.float32)

        def fill_ones(i, c):
            onesbuf[i, :] = ones
            return c

        lax.fori_loop(0, _RPT, fill_ones, 0)
        for h in hh:
            h.wait()
        hh = [pltpu.async_copy(
            dst_hbm.at[pl.ds(ebase + j * _CHUNK, _CHUNK)], didx[j], sem_i)
            for j in range(_NCH)]
        pltpu.sync_copy(onesbuf, s_deg.at[rows])
        for h in hh:
            h.wait()
        plsc.subcore_barrier()

        hh = [pltpu.async_copy(onesbuf, s_deg.at[didx[j]], sem_s, add=True)
              for j in range(_NCH)]
        for h in hh:
            h.wait()
        plsc.subcore_barrier()

        pltpu.sync_copy(s_deg.at[rows], disbuf)

        def mk_dis(i, c):
            d = disbuf[i, :]
            bits = lax.bitcast_convert_type(d, jnp.int32)
            bits = jnp.int32(0x5F3759DF) - lax.shift_right_logical(bits, 1)
            y = lax.bitcast_convert_type(bits, jnp.float32)
            half = 0.5 * d
            y = y * (1.5 - half * y * y)
            y = y * (1.5 - half * y * y)
            y = y * (1.5 - half * y * y)
            disbuf[i, :] = y
            hbuf[i, :] = hbuf[i, :] * y
            return c

        lax.fori_loop(0, _RPT, mk_dis, 0)
        pltpu.sync_copy(hbuf, s_h.at[rows])
        pltpu.sync_copy(hbuf, s_acc.at[rows])
        plsc.subcore_barrier()

        def prop_round(s_to):
            gh = [pltpu.async_copy(
                s_h.at[sidx[j]],
                rowbig.at[pl.ds(j * _CHUNK, _CHUNK)], sem_g)
                for j in range(_NCH)]
            sh = []
            for j in range(_NCH):
                gh[j].wait()
                sh.append(pltpu.async_copy(
                    rowbig.at[pl.ds(j * _CHUNK, _CHUNK)],
                    s_to.at[didx[j]], sem_s, add=True))
            for h in sh:
                h.wait()

        prop_round(s_acc)
        plsc.subcore_barrier()

        pltpu.sync_copy(s_acc.at[rows], accbuf)
        b1v = b1buf[:]

        def mk_h1(i, c):
            a = accbuf[i, :] * disbuf[i, :] + b1v
            a = jnp.maximum(a, 0.0)
            accbuf[i, :] = a * disbuf[i, :]
            return c

        lax.fori_loop(0, _RPT, mk_h1, 0)
        pltpu.sync_copy(accbuf, s_h.at[rows])
        pltpu.sync_copy(accbuf, s_acc2.at[rows])
        plsc.subcore_barrier()

        prop_round(s_acc2)
        plsc.subcore_barrier()

        pltpu.sync_copy(s_acc2.at[rows], accbuf)

        def mk_out(i, c):
            accbuf[i, :] = accbuf[i, :] * disbuf[i, :]
            return c

        lax.fori_loop(0, _RPT, mk_out, 0)
        pltpu.sync_copy(accbuf, out_hbm.at[rows])


_sc_prop = functools.partial(
    pl.kernel,
    mesh=plsc.VectorSubcoreMesh(core_axis_name="c", subcore_axis_name="s"),
    compiler_params=pltpu.CompilerParams(use_tc_tiling_on_sc=False),
    out_type=jax.ShapeDtypeStruct((_N, _HID), jnp.float32),
    scratch_types=(
        [pltpu.VMEM((_CHUNK,), jnp.int32) for _ in range(2 * _NCH)] + [
            pltpu.VMEM((_EPT, _HID), jnp.float32),
            pltpu.VMEM((_RPT, _HID), jnp.float32),
            pltpu.VMEM((_RPT, _HID), jnp.float32),
            pltpu.VMEM((_RPT, _HID), jnp.float32),
            pltpu.VMEM((_RPT, _HID), jnp.float32),
            pltpu.VMEM((_HID,), jnp.float32),
            pltpu.VMEM_SHARED((_N, _HID), jnp.float32),
            pltpu.VMEM_SHARED((_N, _HID), jnp.float32),
            pltpu.VMEM_SHARED((_N, _HID), jnp.float32),
            pltpu.VMEM_SHARED((_N, _HID), jnp.float32),
            pltpu.SemaphoreType.DMA,
            pltpu.SemaphoreType.DMA,
            pltpu.SemaphoreType.DMA,
        ]),
)(_sc_body)


@jax.jit
def kernel(x, edge_index, W1, b1, W2, b2):
    src = edge_index[0]
    dst = edge_index[1]
    hpre = _mm1(x, W1)
    g = _sc_prop(src, dst, hpre, b1)
    out = _mm2(g, W2, b2.reshape(1, _N))
    return out
